# Initial kernel scaffold; baseline (speedup 1.0000x reference)
import jax
import jax.numpy as jnp
from jax.experimental import pallas as pl


def kernel(aligned_fea, T_wg, T_bg, T_wt, T_bt, T_wp, T_bp, T_ww, T_bw, F_wg, F_bg, F_wt, F_bt, F_wp, F_bp, F_ww, F_bw, wf, bf):
    raise NotImplementedError("write your pallas kernel here")


# speedup vs baseline: 2.2647x; 2.2647x over previous
import functools

import jax
import jax.numpy as jnp
from jax import lax
from jax.experimental import pallas as pl
from jax.experimental.pallas import tpu as pltpu

_CENTER = 2


def _make_body(B, N, C, H, W, Ci, O, PAD):
    HW = H * W
    Hp, Wp = H // 4, W // 4
    S = Hp * Wp
    K = 4 * Ci
    inv_n = 1.0 / float(S)
    C2 = 2 * C

    def _pool4x4(scr):
        s1 = scr[pl.ds(0, HW // 4, stride=4), :]
        for j in range(1, 4):
            s1 = jnp.maximum(s1, scr[pl.ds(j, HW // 4, stride=4), :])
        rows = []
        for ph in range(Hp):
            m = s1[(4 * ph + 0) * Wp:(4 * ph + 1) * Wp, :]
            for dh in range(1, 4):
                m = jnp.maximum(m, s1[(4 * ph + dh) * Wp:(4 * ph + dh + 1) * Wp, :])
            rows.append(m)
        if Hp == 1:
            return rows[0]
        return jnp.concatenate(rows, axis=0)

    def body(x_ref, wxe_ref, bxe_ref, wtc_ref, btc_ref, wwt_ref, wwf_ref,
             bw_ref, wt9_ref, bf_ref, o_ref, pool_scr, big_scr, tap_scr,
             wz_scr):
        r_bf = x_ref[0, _CENTER].astype(jnp.bfloat16)
        th_T_all = lax.dot_general(
            r_bf, wtc_ref[...], (((0,), (0,)), ((), ())),
            preferred_element_type=jnp.float32) + btc_ref[...]

        wz_scr[...] = jnp.zeros((2 * Ci, C2), jnp.bfloat16)
        for n in range(N):
            xn = x_ref[0, n].astype(jnp.bfloat16)
            conv = lax.dot_general(
                xn, wxe_ref[n], (((0,), (0,)), ((), ())),
                preferred_element_type=jnp.float32) + bxe_ref[n]
            pool_scr[...] = conv[:, :K]
            theta_F = conv[:, K:K + Ci].astype(jnp.bfloat16)
            xT = conv[:, K + Ci:]
            xx = jnp.concatenate([xT, xT], axis=-1)

            pooled = _pool4x4(pool_scr)
            g_F = pooled[:, 0 * Ci:1 * Ci] * inv_n
            p_F = pooled[:, 1 * Ci:2 * Ci]
            g_T = pooled[:, 2 * Ci:3 * Ci] * inv_n
            p_T = pooled[:, 3 * Ci:4 * Ci]

            m_T = lax.dot_general(p_T, g_T, (((0,), (0,)), ((), ())),
                                  preferred_element_type=jnp.float32)
            m_F = lax.dot_general(p_F, g_F, (((0,), (0,)), ((), ())),
                                  preferred_element_type=jnp.float32)
            a_T = jnp.dot(m_T, wwt_ref[n], preferred_element_type=jnp.float32)
            a_F = jnp.dot(m_F, wwf_ref[n], preferred_element_type=jnp.float32)
            wz_scr[0:Ci, 0:C] = a_T.astype(jnp.bfloat16)
            wz_scr[Ci:2 * Ci, C:C2] = a_F.astype(jnp.bfloat16)

            th = jnp.concatenate(
                [th_T_all[:, n * Ci:(n + 1) * Ci].astype(jnp.bfloat16),
                 theta_F], axis=-1)
            z = jnp.dot(th, wz_scr[...],
                        preferred_element_type=jnp.float32) + bw_ref[n] + xx
            big_scr[:, n * C2:(n + 1) * C2] = z.astype(jnp.bfloat16)

        tap_scr[0:PAD, :] = jnp.zeros((PAD, 9 * O), jnp.float32)
        tap_scr[PAD + HW:, :] = jnp.zeros((PAD, 9 * O), jnp.float32)
        taps = jnp.dot(big_scr[...], wt9_ref[...],
                       preferred_element_type=jnp.float32)
        tap_scr[pl.ds(PAD, HW), :] = taps

        wpos = lax.broadcasted_iota(jnp.int32, (HW, 1), 0) % W
        mL = wpos > 0
        mR = wpos < W - 1
        acc = jnp.zeros((HW, O), jnp.float32) + bf_ref[...]
        for kh in range(3):
            for kw in range(3):
                t = kh * 3 + kw
                off = (kh - 1) * W + (kw - 1)
                sl = tap_scr[pl.ds(PAD + off, HW), t * O:(t + 1) * O]
                if kw == 0:
                    sl = jnp.where(mL, sl, 0.0)
                elif kw == 2:
                    sl = jnp.where(mR, sl, 0.0)
                acc = acc + sl
        o_ref[0] = acc

    return body


@functools.partial(jax.jit, static_argnames=())
def kernel(aligned_fea, T_wg, T_bg, T_wt, T_bt, T_wp, T_bp, T_ww, T_bw,
           F_wg, F_bg, F_wt, F_bt, F_wp, F_bp, F_ww, F_bw, wf, bf):
    B, N, C, H, W = aligned_fea.shape
    HW = H * W
    Ci = T_wt.shape[-1]
    O = wf.shape[-1]
    S = (H // 4) * (W // 4)
    C2 = 2 * C
    Ct = 2 * N * C
    PAD = ((W + 1 + 7) // 8) * 8

    x = aligned_fea.reshape(B, N, C, HW)

    eye = jnp.broadcast_to(jnp.eye(C, dtype=jnp.float32), (N, C, C))
    wxe = jnp.concatenate([F_wg, F_wp, T_wg, T_wp, F_wt, eye],
                          axis=-1).astype(jnp.bfloat16)
    bxe = jnp.concatenate([F_bg, F_bp, T_bg, T_bp, F_bt,
                           jnp.zeros((N, 1, C), jnp.float32)], axis=-1)

    wtc = jnp.transpose(T_wt, (1, 0, 2)).reshape(C, N * Ci).astype(jnp.bfloat16)
    btc = T_bt.reshape(1, N * Ci)

    bw = jnp.concatenate([T_bw, F_bw], axis=-1)

    j = jnp.arange(Ct)
    n_i, s_i, c_i = j // C2, (j % C2) // C, j % C
    perm = s_i * (N * C) + n_i * C + c_i
    wf9 = wf[:, :, perm, :].reshape(9, Ct, O)
    wt9 = jnp.transpose(wf9, (1, 0, 2)).reshape(Ct, 9 * O).astype(jnp.bfloat16)

    body = _make_body(B, N, C, H, W, Ci, O, PAD)

    flops = B * N * (2 * HW * C * (5 * Ci + C) + 2 * HW * C * Ci
                     + 2 * HW * 2 * Ci * 2 * C) + B * 2 * HW * Ct * 9 * O
    bytes_acc = 4 * (B * N * HW * C + B * HW * O) + 2 * (
        wxe.size + wtc.size + wt9.size)

    out = pl.pallas_call(
        body,
        out_shape=jax.ShapeDtypeStruct((B, HW, O), jnp.float32),
        grid=(B,),
        in_specs=[
            pl.BlockSpec((1, N, C, HW), lambda b: (b, 0, 0, 0)),
            pl.BlockSpec((N, C, 5 * Ci + C), lambda b: (0, 0, 0)),
            pl.BlockSpec((N, 1, 5 * Ci + C), lambda b: (0, 0, 0)),
            pl.BlockSpec((C, N * Ci), lambda b: (0, 0)),
            pl.BlockSpec((1, N * Ci), lambda b: (0, 0)),
            pl.BlockSpec((N, Ci, C), lambda b: (0, 0, 0)),
            pl.BlockSpec((N, Ci, C), lambda b: (0, 0, 0)),
            pl.BlockSpec((N, 1, C2), lambda b: (0, 0, 0)),
            pl.BlockSpec((Ct, 9 * O), lambda b: (0, 0)),
            pl.BlockSpec((1, O), lambda b: (0, 0)),
        ],
        out_specs=pl.BlockSpec((1, HW, O), lambda b: (b, 0, 0)),
        scratch_shapes=[
            pltpu.VMEM((HW, 4 * Ci), jnp.float32),
            pltpu.VMEM((HW, Ct), jnp.bfloat16),
            pltpu.VMEM((HW + 2 * PAD, 9 * O), jnp.float32),
            pltpu.VMEM((2 * Ci, C2), jnp.bfloat16),
        ],
        compiler_params=pltpu.CompilerParams(
            dimension_semantics=("parallel",)),
        cost_estimate=pl.CostEstimate(flops=flops, transcendentals=0,
                                      bytes_accessed=bytes_acc),
    )(x, wxe, bxe, wtc, btc, T_ww, F_ww, bw, wt9, bf)

    return jnp.transpose(out.reshape(B, H, W, O), (0, 3, 1, 2))


# speedup vs baseline: 2.3069x; 1.0186x over previous
import functools

import jax
import jax.numpy as jnp
from jax import lax
from jax.experimental import pallas as pl
from jax.experimental.pallas import tpu as pltpu

_CENTER = 2


def _make_body(B, N, C, H, W, Ci, O, PAD):
    HW = H * W
    Hp, Wp = H // 4, W // 4
    S = Hp * Wp
    K = 4 * Ci
    inv_n = 1.0 / float(S)
    C2 = 2 * C

    def _pool4x4(scr, sl):
        s1 = scr[sl, pl.ds(0, HW // 4, stride=4), :]
        for j in range(1, 4):
            s1 = jnp.maximum(s1, scr[sl, pl.ds(j, HW // 4, stride=4), :])
        rows = []
        for ph in range(Hp):
            m = s1[(4 * ph + 0) * Wp:(4 * ph + 1) * Wp, :]
            for dh in range(1, 4):
                m = jnp.maximum(m, s1[(4 * ph + dh) * Wp:(4 * ph + dh + 1) * Wp, :])
            rows.append(m)
        if Hp == 1:
            return rows[0]
        return jnp.concatenate(rows, axis=0)

    def body(x_ref, wxe_ref, bxe_ref, wtc_ref, btc_ref, wwt_ref, wwf_ref,
             bw_ref, wt9_ref, bf_ref, o_ref, pool_scr, big_scr, tap_scr,
             wz_scr):
        r_bf = x_ref[0, _CENTER].astype(jnp.bfloat16)
        th_T_all = lax.dot_general(
            r_bf, wtc_ref[...], (((0,), (0,)), ((), ())),
            preferred_element_type=jnp.float32) + btc_ref[...]

        wz_scr[...] = jnp.zeros((2, 2 * Ci, C2), jnp.bfloat16)
        for n in range(N):
            sl = n % 2
            xn = x_ref[0, n].astype(jnp.bfloat16)
            conv = lax.dot_general(
                xn, wxe_ref[n], (((0,), (0,)), ((), ())),
                preferred_element_type=jnp.float32) + bxe_ref[n]
            pool_scr[sl] = conv[:, :K]
            theta_F = conv[:, K:K + Ci].astype(jnp.bfloat16)
            xT = conv[:, K + Ci:]
            xx = jnp.concatenate([xT, xT], axis=-1)

            pooled = _pool4x4(pool_scr, sl)
            g_F = pooled[:, 0 * Ci:1 * Ci] * inv_n
            p_F = pooled[:, 1 * Ci:2 * Ci]
            g_T = pooled[:, 2 * Ci:3 * Ci] * inv_n
            p_T = pooled[:, 3 * Ci:4 * Ci]

            m_T = lax.dot_general(p_T, g_T, (((0,), (0,)), ((), ())),
                                  preferred_element_type=jnp.float32)
            m_F = lax.dot_general(p_F, g_F, (((0,), (0,)), ((), ())),
                                  preferred_element_type=jnp.float32)
            a_T = jnp.dot(m_T, wwt_ref[n], preferred_element_type=jnp.float32)
            a_F = jnp.dot(m_F, wwf_ref[n], preferred_element_type=jnp.float32)
            wz_scr[sl, 0:Ci, 0:C] = a_T.astype(jnp.bfloat16)
            wz_scr[sl, Ci:2 * Ci, C:C2] = a_F.astype(jnp.bfloat16)

            th = jnp.concatenate(
                [th_T_all[:, n * Ci:(n + 1) * Ci].astype(jnp.bfloat16),
                 theta_F], axis=-1)
            z = jnp.dot(th, wz_scr[sl],
                        preferred_element_type=jnp.float32) + bw_ref[n] + xx
            zb = z.astype(jnp.bfloat16)
            big_scr[:, n * C:(n + 1) * C] = zb[:, :C]
            big_scr[:, N * C + n * C:N * C + (n + 1) * C] = zb[:, C:]

        tap_scr[0:PAD, :] = jnp.zeros((PAD, 9 * O), jnp.float32)
        tap_scr[PAD + HW:, :] = jnp.zeros((PAD, 9 * O), jnp.float32)
        taps = jnp.dot(big_scr[...], wt9_ref[...],
                       preferred_element_type=jnp.float32)
        tap_scr[pl.ds(PAD, HW), :] = taps

        wpos = lax.broadcasted_iota(jnp.int32, (HW, 1), 0) % W
        mL = wpos > 0
        mR = wpos < W - 1
        acc = jnp.zeros((HW, O), jnp.float32) + bf_ref[...]
        for kh in range(3):
            for kw in range(3):
                t = kh * 3 + kw
                off = (kh - 1) * W + (kw - 1)
                sl = tap_scr[pl.ds(PAD + off, HW), t * O:(t + 1) * O]
                if kw == 0:
                    sl = jnp.where(mL, sl, 0.0)
                elif kw == 2:
                    sl = jnp.where(mR, sl, 0.0)
                acc = acc + sl
        o_ref[0] = acc.T

    return body


@functools.partial(jax.jit, static_argnames=())
def kernel(aligned_fea, T_wg, T_bg, T_wt, T_bt, T_wp, T_bp, T_ww, T_bw,
           F_wg, F_bg, F_wt, F_bt, F_wp, F_bp, F_ww, F_bw, wf, bf):
    B, N, C, H, W = aligned_fea.shape
    HW = H * W
    Ci = T_wt.shape[-1]
    O = wf.shape[-1]
    S = (H // 4) * (W // 4)
    C2 = 2 * C
    Ct = 2 * N * C
    PAD = ((W + 1 + 7) // 8) * 8

    x = aligned_fea.reshape(B, N, C, HW)

    eye = jnp.broadcast_to(jnp.eye(C, dtype=jnp.float32), (N, C, C))
    wxe = jnp.concatenate([F_wg, F_wp, T_wg, T_wp, F_wt, eye],
                          axis=-1).astype(jnp.bfloat16)
    bxe = jnp.concatenate([F_bg, F_bp, T_bg, T_bp, F_bt,
                           jnp.zeros((N, 1, C), jnp.float32)], axis=-1)

    wtc = jnp.transpose(T_wt, (1, 0, 2)).reshape(C, N * Ci).astype(jnp.bfloat16)
    btc = T_bt.reshape(1, N * Ci)

    bw = jnp.concatenate([T_bw, F_bw], axis=-1)

    wt9 = jnp.transpose(wf.reshape(9, Ct, O),
                        (1, 0, 2)).reshape(Ct, 9 * O).astype(jnp.bfloat16)

    body = _make_body(B, N, C, H, W, Ci, O, PAD)

    flops = B * N * (2 * HW * C * (5 * Ci + C) + 2 * HW * C * Ci
                     + 2 * HW * 2 * Ci * 2 * C) + B * 2 * HW * Ct * 9 * O
    bytes_acc = 4 * (B * N * HW * C + B * HW * O) + 2 * (
        wxe.size + wtc.size + wt9.size)

    out = pl.pallas_call(
        body,
        out_shape=jax.ShapeDtypeStruct((B, O, HW), jnp.float32),
        grid=(B,),
        in_specs=[
            pl.BlockSpec((1, N, C, HW), lambda b: (b, 0, 0, 0)),
            pl.BlockSpec((N, C, 5 * Ci + C), lambda b: (0, 0, 0)),
            pl.BlockSpec((N, 1, 5 * Ci + C), lambda b: (0, 0, 0)),
            pl.BlockSpec((C, N * Ci), lambda b: (0, 0)),
            pl.BlockSpec((1, N * Ci), lambda b: (0, 0)),
            pl.BlockSpec((N, Ci, C), lambda b: (0, 0, 0)),
            pl.BlockSpec((N, Ci, C), lambda b: (0, 0, 0)),
            pl.BlockSpec((N, 1, C2), lambda b: (0, 0, 0)),
            pl.BlockSpec((Ct, 9 * O), lambda b: (0, 0)),
            pl.BlockSpec((1, O), lambda b: (0, 0)),
        ],
        out_specs=pl.BlockSpec((1, O, HW), lambda b: (b, 0, 0)),
        scratch_shapes=[
            pltpu.VMEM((2, HW, 4 * Ci), jnp.float32),
            pltpu.VMEM((HW, Ct), jnp.bfloat16),
            pltpu.VMEM((HW + 2 * PAD, 9 * O), jnp.float32),
            pltpu.VMEM((2, 2 * Ci, C2), jnp.bfloat16),
        ],
        compiler_params=pltpu.CompilerParams(
            dimension_semantics=("parallel",)),
        cost_estimate=pl.CostEstimate(flops=flops, transcendentals=0,
                                      bytes_accessed=bytes_acc),
    )(x, wxe, bxe, wtc, btc, T_ww, F_ww, bw, wt9, bf)

    return out.reshape(B, O, H, W)
